# baseline (speedup 1.0000x reference)
import functools

import jax
import jax.numpy as jnp
from jax import lax
from jax.experimental import pallas as pl
from jax.experimental.pallas import tpu as pltpu
from jax.experimental.pallas import tpu_sc as plsc

_B = 4096
_L = 200
_EMBED = 64
_NC = 2
_NS = 16
_NW = _NC * _NS
_DPW = _B // _NW
_C0 = 128
_C1 = _L - _C0

_RB = 256
_NBLK = 999936 // _RB
_TAILR = 40
_PACKR = _NBLK * (_RB // 2) + _TAILR
_VCAP = 2 * _PACKR


def _repack_body(tt_hbm, tail_hbm, out_hbm, inb0, inb1, outb0, outb1,
                 sem_i, sem_o):
    wid = lax.axis_index("s") * _NC + lax.axis_index("c")
    nb = 122 + jnp.where(wid < (_NBLK - 122 * _NW), 1, 0)
    last_b = wid + _NW * (nb - 1)

    lanes = lax.iota(jnp.int32, 16)

    def start_in(b, buf):
        pltpu.make_async_copy(
            tt_hbm.at[:, pl.ds(b * _RB, _RB)], buf, sem_i).start()

    def wait_in(buf):
        pltpu.make_async_copy(
            tt_hbm.at[:, pl.ds(0, _RB)], buf, sem_i).wait()

    def wait_out_one():
        pltpu.make_async_copy(
            outb0, out_hbm.at[pl.ds(0, _RB // 2)], sem_o).wait()

    def transpose(buf, obuf):
        def q_body(q, _):
            for h in range(2):
                r_idx = jnp.zeros((16,), jnp.int32) + (2 * q + h)
                for g in range(4):
                    c_idx = lanes + 16 * g
                    v = plsc.load_gather(buf, [c_idx, r_idx])
                    obuf[q, pl.ds(64 * h + 16 * g, 16)] = v
            return 0

        lax.fori_loop(0, _RB // 2, q_body, 0)

    start_in(wid, inb0)

    def body(t, _):
        b = wid + _NW * t
        bn = jnp.minimum(b + _NW, last_b)
        buf_sel = t % 2

        @pl.when(buf_sel == 0)
        def _():
            start_in(bn, inb1)
            wait_in(inb0)

            @pl.when(t >= 2)
            def _():
                wait_out_one()
            transpose(inb0, outb0)
            pltpu.make_async_copy(
                outb0, out_hbm.at[pl.ds(b * (_RB // 2), _RB // 2)],
                sem_o).start()

        @pl.when(buf_sel == 1)
        def _():
            start_in(bn, inb0)
            wait_in(inb1)

            @pl.when(t >= 2)
            def _():
                wait_out_one()
            transpose(inb1, outb1)
            pltpu.make_async_copy(
                outb1, out_hbm.at[pl.ds(b * (_RB // 2), _RB // 2)],
                sem_o).start()
        return 0

    lax.fori_loop(0, nb, body, 0)
    wait_in(inb0)
    wait_out_one()
    wait_out_one()

    @pl.when(wid == 2)
    def _():
        pltpu.sync_copy(tail_hbm, outb0.at[pl.ds(0, _TAILR)])
        pltpu.sync_copy(outb0.at[pl.ds(0, _TAILR)],
                        out_hbm.at[pl.ds(_NBLK * (_RB // 2), _TAILR)])


def _repack(tt, tail):
    mesh = plsc.VectorSubcoreMesh(core_axis_name="c", subcore_axis_name="s")
    f = pl.kernel(
        _repack_body,
        out_type=jax.ShapeDtypeStruct((_PACKR, 2 * _EMBED), jnp.float32),
        mesh=mesh,
        scratch_types=[
            pltpu.VMEM((_EMBED, _RB), jnp.float32),
            pltpu.VMEM((_EMBED, _RB), jnp.float32),
            pltpu.VMEM((_RB // 2, 2 * _EMBED), jnp.float32),
            pltpu.VMEM((_RB // 2, 2 * _EMBED), jnp.float32),
            pltpu.SemaphoreType.DMA,
            pltpu.SemaphoreType.DMA,
        ],
        compiler_params=pltpu.CompilerParams(use_tc_tiling_on_sc=True,
                                             needs_layout_passes=False),
    )
    return f(tt, tail)


def _pool_body(x_hbm, table_hbm, out_hbm, idx_v, rows0, rows1, out_v,
               sem0, sem1):
    wid = lax.axis_index("s") * _NC + lax.axis_index("c")
    base = wid * _DPW

    pltpu.sync_copy(x_hbm.at[pl.ds(base, _DPW)], idx_v)

    def start(d, rows, sem):
        pltpu.make_async_copy(
            table_hbm.at[idx_v.at[d, pl.ds(0, _C0)]],
            rows.at[pl.ds(0, _C0)], sem).start()
        pltpu.make_async_copy(
            table_hbm.at[idx_v.at[d, pl.ds(_C0, _C1)]],
            rows.at[pl.ds(_C0, _C1)], sem).start()

    def wait(rows, sem):
        pltpu.make_async_copy(table_hbm.at[pl.ds(0, _L)], rows, sem).wait()

    def accum(rows, d):
        def row_body(i, acc):
            return tuple(
                acc[c] + rows[i, pl.ds(16 * c, 16)] for c in range(4))

        zero = jnp.zeros((16,), jnp.float32)
        acc = lax.fori_loop(0, _L, row_body, (zero, zero, zero, zero),
                            unroll=8)
        for c in range(4):
            out_v[d, pl.ds(16 * c, 16)] = acc[c]

    start(0, rows0, sem0)

    def body(t, _):
        d0 = 2 * t
        d1 = d0 + 1
        start(d1, rows1, sem1)
        wait(rows0, sem0)
        accum(rows0, d0)
        d2 = jnp.minimum(d0 + 2, _DPW - 1)
        start(d2, rows0, sem0)
        wait(rows1, sem1)
        accum(rows1, d1)
        return 0

    lax.fori_loop(0, _DPW // 2, body, 0)
    wait(rows0, sem0)

    pltpu.sync_copy(out_v, out_hbm.at[pl.ds(base, _DPW)])


def _pool(x, table):
    mesh = plsc.VectorSubcoreMesh(core_axis_name="c", subcore_axis_name="s")
    f = pl.kernel(
        _pool_body,
        out_type=jax.ShapeDtypeStruct((_B, _EMBED), jnp.float32),
        mesh=mesh,
        scratch_types=[
            pltpu.VMEM((_DPW, _L), jnp.int32),
            pltpu.VMEM((_L, _EMBED), jnp.float32),
            pltpu.VMEM((_L, _EMBED), jnp.float32),
            pltpu.VMEM((_DPW, _EMBED), jnp.float32),
            pltpu.SemaphoreType.DMA,
            pltpu.SemaphoreType.DMA,
        ],
        compiler_params=pltpu.CompilerParams(use_tc_tiling_on_sc=False),
    )
    return f(x, table)


def _mlp_body(s_ref, w1_ref, b1_ref, w2_ref, b2_ref, o_ref):
    s = s_ref[...]
    norm = jnp.sqrt(jnp.sum(s * s, axis=1, keepdims=True))
    ns = s / jnp.maximum(norm, 1e-12)
    h = lax.dot_general(ns, w1_ref[...], (((1,), (1,)), ((), ())),
                        preferred_element_type=jnp.float32)
    h = jnp.maximum(h + b1_ref[...], 0.0)
    o = lax.dot_general(h, w2_ref[...], (((1,), (1,)), ((), ())),
                        preferred_element_type=jnp.float32)
    o = o + b2_ref[...]
    m = jnp.max(o, axis=1, keepdims=True)
    e = jnp.exp(o - m)
    o_ref[...] = e / jnp.sum(e, axis=1, keepdims=True)


def _mlp(s, W1, b1, W2, b2):
    blk = 512
    grid = _B // blk
    return pl.pallas_call(
        _mlp_body,
        grid=(grid,),
        in_specs=[
            pl.BlockSpec((blk, _EMBED), lambda i: (i, 0)),
            pl.BlockSpec(W1.shape, lambda i: (0, 0)),
            pl.BlockSpec((1, W1.shape[0]), lambda i: (0, 0)),
            pl.BlockSpec(W2.shape, lambda i: (0, 0)),
            pl.BlockSpec((1, W2.shape[0]), lambda i: (0, 0)),
        ],
        out_specs=pl.BlockSpec((blk, _EMBED), lambda i: (i, 0)),
        out_shape=jax.ShapeDtypeStruct((_B, _EMBED), jnp.float32),
    )(s, W1, b1, W2, b2)


def kernel(x, table, W1, b1, W2, b2):
    x = x.astype(jnp.int32)
    tt = jnp.swapaxes(table, 0, 1)
    t80 = jnp.concatenate(
        [table[_NBLK * _RB:], jnp.zeros((15, _EMBED), jnp.float32)])
    tail = jnp.reshape(t80, (_TAILR, 2 * _EMBED))
    packed = _repack(tt, tail)
    tbl = jnp.reshape(packed, (_VCAP, _EMBED))
    s = _pool(x, tbl)
    return _mlp(s, W1, b1.reshape(1, -1), W2, b2.reshape(1, -1))


# speedup vs baseline: 1.2054x; 1.2054x over previous
import functools

import jax
import jax.numpy as jnp
from jax import lax
from jax.experimental import pallas as pl
from jax.experimental.pallas import tpu as pltpu
from jax.experimental.pallas import tpu_sc as plsc

_B = 4096
_L = 200
_EMBED = 64
_NC = 2
_NS = 16
_NW = _NC * _NS
_DPW = _B // _NW
_C0 = 128
_C1 = _L - _C0

_RB = 256
_NBLK = 999936 // _RB
_TAILR = 40
_PACKR = _NBLK * (_RB // 2) + _TAILR
_VCAP = 2 * _PACKR


_BW = _EMBED * _RB


def _repack_body(tt_hbm, tail_hbm, out_hbm, inb0, inb1, outb0, outb1,
                 sem_i, sem_o):
    wid = lax.axis_index("s") * _NC + lax.axis_index("c")
    nb = 122 + jnp.where(wid < (_NBLK - 122 * _NW), 1, 0)
    last_b = wid + _NW * (nb - 1)

    lanes = lax.iota(jnp.int32, 16)
    addrlane = (lanes // 2) * 128 + 64 * (lanes % 2)

    def start_in(b, buf):
        pltpu.make_async_copy(
            tt_hbm.at[:, pl.ds(b * _RB, _RB)], buf, sem_i).start()

    def wait_in(buf):
        pltpu.make_async_copy(
            tt_hbm.at[:, pl.ds(0, _RB)], buf, sem_i).wait()

    def wait_out_one():
        pltpu.make_async_copy(
            outb0, out_hbm.at[pl.ds(0, _BW)], sem_o).wait()

    def transpose(buf, obuf):
        def c_body(c, _):
            addr_c = addrlane + c
            for r0 in range(0, _RB, 16):
                v = buf[c, pl.ds(r0, 16)]
                plsc.store_scatter(obuf, [addr_c + r0 * 64], v)
            return 0

        lax.fori_loop(0, _EMBED, c_body, 0)

    start_in(wid, inb0)

    def body(t, _):
        b = wid + _NW * t
        bn = jnp.minimum(b + _NW, last_b)
        buf_sel = t % 2

        @pl.when(buf_sel == 0)
        def _():
            start_in(bn, inb1)
            wait_in(inb0)

            @pl.when(t >= 2)
            def _():
                wait_out_one()
            transpose(inb0, outb0)
            pltpu.make_async_copy(
                outb0, out_hbm.at[pl.ds(b * _BW, _BW)], sem_o).start()

        @pl.when(buf_sel == 1)
        def _():
            start_in(bn, inb0)
            wait_in(inb1)

            @pl.when(t >= 2)
            def _():
                wait_out_one()
            transpose(inb1, outb1)
            pltpu.make_async_copy(
                outb1, out_hbm.at[pl.ds(b * _BW, _BW)], sem_o).start()
        return 0

    lax.fori_loop(0, nb, body, 0)
    wait_in(inb0)
    wait_out_one()
    wait_out_one()

    @pl.when(wid == 2)
    def _():
        pltpu.sync_copy(tail_hbm, outb0.at[pl.ds(0, _TAILR * 2 * _EMBED)])
        pltpu.sync_copy(outb0.at[pl.ds(0, _TAILR * 2 * _EMBED)],
                        out_hbm.at[pl.ds(_NBLK * _BW, _TAILR * 2 * _EMBED)])


def _repack(tt, tail):
    mesh = plsc.VectorSubcoreMesh(core_axis_name="c", subcore_axis_name="s")
    f = pl.kernel(
        _repack_body,
        out_type=jax.ShapeDtypeStruct((_PACKR * 2 * _EMBED,), jnp.float32),
        mesh=mesh,
        scratch_types=[
            pltpu.VMEM((_EMBED, _RB), jnp.float32),
            pltpu.VMEM((_EMBED, _RB), jnp.float32),
            pltpu.VMEM((_BW,), jnp.float32),
            pltpu.VMEM((_BW,), jnp.float32),
            pltpu.SemaphoreType.DMA,
            pltpu.SemaphoreType.DMA,
        ],
        compiler_params=pltpu.CompilerParams(use_tc_tiling_on_sc=True,
                                             needs_layout_passes=False),
    )
    return f(tt, tail)


def _pool_body(x_hbm, table_hbm, out_hbm, idx_v, rows0, rows1, out_v,
               sem0, sem1):
    wid = lax.axis_index("s") * _NC + lax.axis_index("c")
    base = wid * _DPW

    pltpu.sync_copy(x_hbm.at[pl.ds(base, _DPW)], idx_v)

    def start(d, rows, sem):
        pltpu.make_async_copy(
            table_hbm.at[idx_v.at[d, pl.ds(0, _C0)]],
            rows.at[pl.ds(0, _C0)], sem).start()
        pltpu.make_async_copy(
            table_hbm.at[idx_v.at[d, pl.ds(_C0, _C1)]],
            rows.at[pl.ds(_C0, _C1)], sem).start()

    def wait(rows, sem):
        pltpu.make_async_copy(table_hbm.at[pl.ds(0, _L)], rows, sem).wait()

    def accum(rows, d):
        def row_body(i, acc):
            return tuple(
                acc[c] + rows[i, pl.ds(16 * c, 16)] for c in range(4))

        zero = jnp.zeros((16,), jnp.float32)
        acc = lax.fori_loop(0, _L, row_body, (zero, zero, zero, zero),
                            unroll=8)
        for c in range(4):
            out_v[d, pl.ds(16 * c, 16)] = acc[c]

    start(0, rows0, sem0)

    def body(t, _):
        d0 = 2 * t
        d1 = d0 + 1
        start(d1, rows1, sem1)
        wait(rows0, sem0)
        accum(rows0, d0)
        d2 = jnp.minimum(d0 + 2, _DPW - 1)
        start(d2, rows0, sem0)
        wait(rows1, sem1)
        accum(rows1, d1)
        return 0

    lax.fori_loop(0, _DPW // 2, body, 0)
    wait(rows0, sem0)

    pltpu.sync_copy(out_v, out_hbm.at[pl.ds(base, _DPW)])


def _pool(x, table):
    mesh = plsc.VectorSubcoreMesh(core_axis_name="c", subcore_axis_name="s")
    f = pl.kernel(
        _pool_body,
        out_type=jax.ShapeDtypeStruct((_B, _EMBED), jnp.float32),
        mesh=mesh,
        scratch_types=[
            pltpu.VMEM((_DPW, _L), jnp.int32),
            pltpu.VMEM((_L, _EMBED), jnp.float32),
            pltpu.VMEM((_L, _EMBED), jnp.float32),
            pltpu.VMEM((_DPW, _EMBED), jnp.float32),
            pltpu.SemaphoreType.DMA,
            pltpu.SemaphoreType.DMA,
        ],
        compiler_params=pltpu.CompilerParams(use_tc_tiling_on_sc=False),
    )
    return f(x, table)


def _mlp_body(s_ref, w1_ref, b1_ref, w2_ref, b2_ref, o_ref):
    s = s_ref[...]
    norm = jnp.sqrt(jnp.sum(s * s, axis=1, keepdims=True))
    ns = s / jnp.maximum(norm, 1e-12)
    h = lax.dot_general(ns, w1_ref[...], (((1,), (1,)), ((), ())),
                        preferred_element_type=jnp.float32)
    h = jnp.maximum(h + b1_ref[...], 0.0)
    o = lax.dot_general(h, w2_ref[...], (((1,), (1,)), ((), ())),
                        preferred_element_type=jnp.float32)
    o = o + b2_ref[...]
    m = jnp.max(o, axis=1, keepdims=True)
    e = jnp.exp(o - m)
    o_ref[...] = e / jnp.sum(e, axis=1, keepdims=True)


def _mlp(s, W1, b1, W2, b2):
    blk = 512
    grid = _B // blk
    return pl.pallas_call(
        _mlp_body,
        grid=(grid,),
        in_specs=[
            pl.BlockSpec((blk, _EMBED), lambda i: (i, 0)),
            pl.BlockSpec(W1.shape, lambda i: (0, 0)),
            pl.BlockSpec((1, W1.shape[0]), lambda i: (0, 0)),
            pl.BlockSpec(W2.shape, lambda i: (0, 0)),
            pl.BlockSpec((1, W2.shape[0]), lambda i: (0, 0)),
        ],
        out_specs=pl.BlockSpec((blk, _EMBED), lambda i: (i, 0)),
        out_shape=jax.ShapeDtypeStruct((_B, _EMBED), jnp.float32),
    )(s, W1, b1, W2, b2)


def kernel(x, table, W1, b1, W2, b2):
    x = x.astype(jnp.int32)
    tt = jnp.swapaxes(table, 0, 1)
    t80 = jnp.concatenate(
        [table[_NBLK * _RB:], jnp.zeros((15, _EMBED), jnp.float32)])
    tail = jnp.reshape(t80, (_TAILR * 2 * _EMBED,))
    packed = _repack(tt, tail)
    tbl = jnp.reshape(packed, (_VCAP, _EMBED))
    s = _pool(x, tbl)
    return _mlp(s, W1, b1.reshape(1, -1), W2, b2.reshape(1, -1))
